# baseline (speedup 1.0000x reference)
import functools

import jax
import jax.numpy as jnp
from jax import lax
from jax.experimental import pallas as pl
from jax.experimental.pallas import tpu as pltpu
from jax.experimental.pallas import tpu_sc as plsc

N = 10000
D = 128
E = 320000

NC = 2
NS = 16
NW = NC * NS
CHUNK = 64
CH = 160
EP = NW * CH * CHUNK
NACC = 10080
PT = NACC // NS


@functools.lru_cache(maxsize=None)
def _make_sc_agg(width):
    mesh = plsc.VectorSubcoreMesh(core_axis_name="c", subcore_axis_name="s")

    @functools.partial(
        pl.kernel,
        out_type=jax.ShapeDtypeStruct((NC, NACC, width), jnp.float32),
        mesh=mesh,
        scratch_types=[
            pltpu.VMEM_SHARED((NACC, width), jnp.float32),
            pltpu.VMEM((CH, CHUNK), jnp.int32),
            pltpu.VMEM((CH, CHUNK), jnp.int32),
            pltpu.VMEM((CHUNK, width), jnp.float32),
            pltpu.VMEM((CHUNK, width), jnp.float32),
            pltpu.SemaphoreType.DMA,
            pltpu.SemaphoreType.DMA,
            pltpu.SemaphoreType.DMA,
            pltpu.SemaphoreType.DMA,
        ],
        compiler_params=pltpu.CompilerParams(use_tc_tiling_on_sc=False),
    )
    def sc_agg(table, srcr, dstr, zrows, parts, acc, sidx, didx, buf0, buf1,
               gsem0, gsem1, ssem0, ssem1):
        c = lax.axis_index("c")
        s = lax.axis_index("s")
        wid = s * NC + c
        pltpu.sync_copy(zrows, acc.at[pl.ds(s * PT, PT)])
        pltpu.sync_copy(srcr.at[wid], sidx)
        pltpu.sync_copy(dstr.at[wid], didx)
        plsc.subcore_barrier()

        pltpu.async_copy(table.at[sidx.at[0]], buf0, gsem0)
        pltpu.async_copy(table.at[sidx.at[1]], buf1, gsem1)

        @pl.loop(0, CH - 2, step=2)
        def _(j):
            pltpu.make_async_copy(table.at[sidx.at[j]], buf0, gsem0).wait()
            pltpu.sync_copy(buf0, acc.at[didx.at[j]], add=True)
            pltpu.async_copy(table.at[sidx.at[j + 2]], buf0, gsem0)
            pltpu.make_async_copy(table.at[sidx.at[j + 1]], buf1, gsem1).wait()
            pltpu.sync_copy(buf1, acc.at[didx.at[j + 1]], add=True)
            pltpu.async_copy(table.at[sidx.at[j + 3]], buf1, gsem1)

        pltpu.make_async_copy(table.at[sidx.at[CH - 2]], buf0, gsem0).wait()
        pltpu.sync_copy(buf0, acc.at[didx.at[CH - 2]], add=True)
        pltpu.make_async_copy(table.at[sidx.at[CH - 1]], buf1, gsem1).wait()
        pltpu.sync_copy(buf1, acc.at[didx.at[CH - 1]], add=True)

        plsc.subcore_barrier()
        pltpu.sync_copy(acc.at[pl.ds(s * PT, PT)],
                        parts.at[c].at[pl.ds(s * PT, PT)])

    return sc_agg


def _xside_body(x, w, b, out_ref):
    out_ref[...] = lax.dot_general(
        x[...], w[...], (((1,), (1,)), ((), ())),
        preferred_element_type=jnp.float32,
        precision=lax.Precision.HIGHEST) + b[...]


def _combine0_body(p0, p1, wla, sel, zr, wr1, br1, h_ref, rdeg_ref, zr1_ref):
    ps = p0[...] + p1[...]
    dn = (((1,), (0,)), ((), ()))
    zl = lax.dot_general(ps, wla[...], dn,
                         preferred_element_type=jnp.float32,
                         precision=lax.Precision.HIGHEST)
    degb = lax.dot_general(ps, sel[...], dn,
                           preferred_element_type=jnp.float32,
                           precision=lax.Precision.HIGHEST)
    rdeg = 1.0 / jnp.maximum(degb, 1.0)
    z = zl * rdeg + zr[...]
    sig = 1.0 / (1.0 + jnp.exp(-z))
    h = z * (0.5 + 0.5 * sig)
    h_ref[...] = h
    rdeg_ref[...] = rdeg
    zr1_ref[...] = lax.dot_general(
        h, wr1[...], (((1,), (1,)), ((), ())),
        preferred_element_type=jnp.float32,
        precision=lax.Precision.HIGHEST) + br1[...]


def _combine1_body(p0, p1, rdeg, wl, zr, out_ref):
    ps = p0[...] + p1[...]
    zl = lax.dot_general(ps, wl[...], (((1,), (1,)), ((), ())),
                         preferred_element_type=jnp.float32,
                         precision=lax.Precision.HIGHEST)
    out_ref[...] = zl * rdeg[...] + zr[...]


_RB = 2000
_GRID = N // _RB


def _row_spec(width):
    return pl.BlockSpec((_RB, width), lambda i: (i, 0))


def _full_spec(shape):
    return pl.BlockSpec(shape, lambda i: tuple(0 for _ in shape))


def _xside(x, w, b):
    return pl.pallas_call(
        _xside_body,
        grid=(_GRID,),
        in_specs=[_row_spec(D), _full_spec((D, D)), _full_spec((1, D))],
        out_specs=_row_spec(D),
        out_shape=jax.ShapeDtypeStruct((N, D), jnp.float32),
    )(x, w, b)


def _combine0(p0, p1, wla, sel, zr, wr1, br1):
    return pl.pallas_call(
        _combine0_body,
        grid=(_GRID,),
        in_specs=[
            _row_spec(D + 16), _row_spec(D + 16),
            _full_spec((D + 16, D)), _full_spec((D + 16, D)),
            _row_spec(D), _full_spec((D, D)), _full_spec((1, D)),
        ],
        out_specs=[_row_spec(D), _row_spec(D), _row_spec(D)],
        out_shape=[jax.ShapeDtypeStruct((N, D), jnp.float32),
                   jax.ShapeDtypeStruct((N, D), jnp.float32),
                   jax.ShapeDtypeStruct((N, D), jnp.float32)],
    )(p0, p1, wla, sel, zr, wr1, br1)


def _combine1(p0, p1, rdeg, wl, zr):
    return pl.pallas_call(
        _combine1_body,
        grid=(_GRID,),
        in_specs=[
            _row_spec(D), _row_spec(D), _row_spec(D),
            _full_spec((D, D)), _row_spec(D),
        ],
        out_specs=_row_spec(D),
        out_shape=jax.ShapeDtypeStruct((N, D), jnp.float32),
    )(p0, p1, rdeg, wl, zr)


def kernel(x, edge_index, W_l0, b_l0, W_r0, W_l1, b_l1, W_r1):
    src = edge_index[0]
    dst = edge_index[1]
    pad = EP - E
    pad_iota = jnp.arange(pad, dtype=jnp.int32)
    srcp = jnp.concatenate([src, pad_iota % N]).reshape(NW, CH, CHUNK)
    dstp = jnp.concatenate([dst, N + pad_iota % (NACC - N)]).reshape(
        NW, CH, CHUNK)

    x_aug = jnp.concatenate([x, jnp.ones((N, 16), jnp.float32)], axis=1)
    parts0 = _make_sc_agg(D + 16)(x_aug, srcp, dstp,
                                  jnp.zeros((PT, D + 16), jnp.float32))
    zr0 = _xside(x, W_r0, b_l0.reshape(1, D))

    wla = jnp.concatenate([W_l0.T, jnp.zeros((16, D), jnp.float32)], axis=0)
    sel = jnp.zeros((D + 16, D), jnp.float32).at[D, :].set(1.0)
    h0, rdeg, zr1 = _combine0(parts0[0], parts0[1], wla, sel, zr0,
                              W_r1, b_l1.reshape(1, D))

    parts1 = _make_sc_agg(D)(h0, srcp, dstp, jnp.zeros((PT, D), jnp.float32))

    return _combine1(parts1[0], parts1[1], rdeg, W_l1, zr1)


# speedup vs baseline: 1.1696x; 1.1696x over previous
import functools

import jax
import jax.numpy as jnp
from jax import lax
from jax.experimental import pallas as pl
from jax.experimental.pallas import tpu as pltpu
from jax.experimental.pallas import tpu_sc as plsc

N = 10000
D = 128
E = 320000

NC = 2
NS = 16
NW = NC * NS
CHUNK = 128
CH = 80
GP = 4
NG = CH // GP
EP = NW * CH * CHUNK
NACC = 10080
PT = NACC // NS


@functools.lru_cache(maxsize=None)
def _make_sc_agg(width):
    mesh = plsc.VectorSubcoreMesh(core_axis_name="c", subcore_axis_name="s")

    @functools.partial(
        pl.kernel,
        out_type=jax.ShapeDtypeStruct((NC, NACC, width), jnp.float32),
        mesh=mesh,
        scratch_types=[
            pltpu.VMEM_SHARED((NACC, width), jnp.float32),
            pltpu.VMEM((2 * GP, CHUNK), jnp.int32),
            pltpu.VMEM((2 * GP, CHUNK), jnp.int32),
            pltpu.VMEM((CHUNK, width), jnp.float32),
            pltpu.VMEM((CHUNK, width), jnp.float32),
            pltpu.SemaphoreType.DMA,
            pltpu.SemaphoreType.DMA,
            pltpu.SemaphoreType.DMA,
            pltpu.SemaphoreType.DMA,
        ],
        compiler_params=pltpu.CompilerParams(use_tc_tiling_on_sc=False),
    )
    def sc_agg(table, srcr, dstr, parts, acc, sidx, didx, buf0, buf1,
               gsem0, gsem1, isem_s, isem_d):
        c = lax.axis_index("c")
        s = lax.axis_index("s")
        wid = s * NC + c
        mysrc = srcr.at[wid]
        mydst = dstr.at[wid]

        @pl.loop(0, CHUNK)
        def _(r):
            for cp in range(width // 16):
                buf0[r, pl.ds(cp * 16, 16)] = jnp.zeros((16,), jnp.float32)

        off = 0
        while off < PT:
            sz = min(CHUNK, PT - off)
            pltpu.sync_copy(buf0.at[pl.ds(0, sz)],
                            acc.at[pl.ds(s * PT + off, sz)])
            off += sz

        pltpu.sync_copy(mysrc.at[pl.ds(0, GP)], sidx.at[pl.ds(0, GP)])
        pltpu.sync_copy(mydst.at[pl.ds(0, GP)], didx.at[pl.ds(0, GP)])
        pltpu.async_copy(mysrc.at[pl.ds(GP, GP)], sidx.at[pl.ds(GP, GP)],
                         isem_s)
        pltpu.async_copy(mydst.at[pl.ds(GP, GP)], didx.at[pl.ds(GP, GP)],
                         isem_d)
        plsc.subcore_barrier()

        pltpu.async_copy(table.at[sidx.at[0]], buf0, gsem0)
        pltpu.async_copy(table.at[sidx.at[1]], buf1, gsem1)

        @pl.loop(0, NG)
        def _(g):
            base = lax.rem(g, 2) * GP
            nbase = lax.rem(g + 1, 2) * GP

            @pl.when(g < NG - 1)
            def _():
                pltpu.make_async_copy(mysrc.at[pl.ds(0, GP)],
                                      sidx.at[pl.ds(0, GP)], isem_s).wait()
                pltpu.make_async_copy(mydst.at[pl.ds(0, GP)],
                                      didx.at[pl.ds(0, GP)], isem_d).wait()

            for k in range(GP):
                buf, gsem = (buf0, gsem0) if k % 2 == 0 else (buf1, gsem1)
                pltpu.make_async_copy(table.at[sidx.at[0]], buf, gsem).wait()
                pltpu.sync_copy(buf, acc.at[didx.at[base + k]], add=True)
                if k < 2:
                    pltpu.async_copy(table.at[sidx.at[base + k + 2]], buf,
                                     gsem)
                else:
                    @pl.when(g < NG - 1)
                    def _():
                        pltpu.async_copy(table.at[sidx.at[nbase + k - 2]],
                                         buf, gsem)

            @pl.when(g < NG - 2)
            def _():
                pltpu.async_copy(mysrc.at[pl.ds((g + 2) * GP, GP)],
                                 sidx.at[pl.ds(base, GP)], isem_s)
                pltpu.async_copy(mydst.at[pl.ds((g + 2) * GP, GP)],
                                 didx.at[pl.ds(base, GP)], isem_d)

        plsc.subcore_barrier()
        pltpu.sync_copy(acc.at[pl.ds(s * PT, PT)],
                        parts.at[c].at[pl.ds(s * PT, PT)])

    return sc_agg


def _xside_body(x, w, b, out_ref):
    out_ref[...] = lax.dot_general(
        x[...], w[...], (((1,), (1,)), ((), ())),
        preferred_element_type=jnp.float32,
        precision=lax.Precision.HIGHEST) + b[...]


def _combine0_body(p0, p1, wla, sel, zr, h_ref, rdeg_ref):
    ps = p0[...] + p1[...]
    dn = (((1,), (0,)), ((), ()))
    zl = lax.dot_general(ps, wla[...], dn,
                         preferred_element_type=jnp.float32,
                         precision=lax.Precision.HIGHEST)
    degb = lax.dot_general(ps, sel[...], dn,
                           preferred_element_type=jnp.float32,
                           precision=lax.Precision.HIGHEST)
    rdeg = 1.0 / jnp.maximum(degb, 1.0)
    z = zl * rdeg + zr[...]
    sig = 1.0 / (1.0 + jnp.exp(-z))
    h_ref[...] = z * (0.5 + 0.5 * sig)
    rdeg_ref[...] = rdeg


def _combine1_body(p0, p1, rdeg, wl, zr, out_ref):
    ps = p0[...] + p1[...]
    zl = lax.dot_general(ps, wl[...], (((1,), (1,)), ((), ())),
                         preferred_element_type=jnp.float32,
                         precision=lax.Precision.HIGHEST)
    out_ref[...] = zl * rdeg[...] + zr[...]


_RB = 2000
_GRID = N // _RB


def _row_spec(width):
    return pl.BlockSpec((_RB, width), lambda i: (i, 0))


def _full_spec(shape):
    return pl.BlockSpec(shape, lambda i: tuple(0 for _ in shape))


def _xside(x, w, b):
    return pl.pallas_call(
        _xside_body,
        grid=(_GRID,),
        in_specs=[_row_spec(D), _full_spec((D, D)), _full_spec((1, D))],
        out_specs=_row_spec(D),
        out_shape=jax.ShapeDtypeStruct((N, D), jnp.float32),
    )(x, w, b)


def _combine0(p0, p1, wla, sel, zr):
    return pl.pallas_call(
        _combine0_body,
        grid=(_GRID,),
        in_specs=[
            _row_spec(D + 16), _row_spec(D + 16),
            _full_spec((D + 16, D)), _full_spec((D + 16, D)),
            _row_spec(D),
        ],
        out_specs=[_row_spec(D), _row_spec(D)],
        out_shape=[jax.ShapeDtypeStruct((N, D), jnp.float32),
                   jax.ShapeDtypeStruct((N, D), jnp.float32)],
    )(p0, p1, wla, sel, zr)


def _combine1(p0, p1, rdeg, wl, zr):
    return pl.pallas_call(
        _combine1_body,
        grid=(_GRID,),
        in_specs=[
            _row_spec(D), _row_spec(D), _row_spec(D),
            _full_spec((D, D)), _row_spec(D),
        ],
        out_specs=_row_spec(D),
        out_shape=jax.ShapeDtypeStruct((N, D), jnp.float32),
    )(p0, p1, rdeg, wl, zr)


def kernel(x, edge_index, W_l0, b_l0, W_r0, W_l1, b_l1, W_r1):
    src = edge_index[0]
    dst = edge_index[1]
    pad = EP - E
    pad_iota = jnp.arange(pad, dtype=jnp.int32)
    srcp = jnp.concatenate([src, pad_iota % N]).reshape(NW, CH, CHUNK)
    dstp = jnp.concatenate([dst, N + pad_iota % (NACC - N)]).reshape(
        NW, CH, CHUNK)

    x_aug = jnp.concatenate([x, jnp.ones((N, 16), jnp.float32)], axis=1)
    parts0 = _make_sc_agg(D + 16)(x_aug, srcp, dstp)
    zr0 = _xside(x, W_r0, b_l0.reshape(1, D))

    wla = jnp.concatenate([W_l0.T, jnp.zeros((16, D), jnp.float32)], axis=0)
    sel = jnp.zeros((D + 16, D), jnp.float32).at[D, :].set(1.0)
    h0, rdeg = _combine0(parts0[0], parts0[1], wla, sel, zr0)

    parts1 = _make_sc_agg(D)(h0, srcp, dstp)
    zr1 = _xside(h0, W_r1, b_l1.reshape(1, D))

    return _combine1(parts1[0], parts1[1], rdeg, W_l1, zr1)


# speedup vs baseline: 1.3133x; 1.1229x over previous
import functools

import jax
import jax.numpy as jnp
from jax import lax
from jax.experimental import pallas as pl
from jax.experimental.pallas import tpu as pltpu
from jax.experimental.pallas import tpu_sc as plsc

N = 10000
D = 128
E = 320000

NC = 2
NS = 16
NW = NC * NS
CHUNK = 128
CH = 80
GP = 4
NG = CH // GP
EP = NW * CH * CHUNK
NACC = 10080
PT = NACC // NS
DW = 16


@functools.lru_cache(maxsize=None)
def _make_sc_agg(with_deg):
    mesh = plsc.VectorSubcoreMesh(core_axis_name="c", subcore_axis_name="s")

    out_type = [jax.ShapeDtypeStruct((NC, NACC, D), jnp.float32)]
    scratch = [
        pltpu.VMEM_SHARED((NACC, D), jnp.float32),
        pltpu.VMEM((2 * GP, CHUNK), jnp.int32),
        pltpu.VMEM((2 * GP, CHUNK), jnp.int32),
        pltpu.VMEM((CHUNK, D), jnp.float32),
        pltpu.VMEM((CHUNK, D), jnp.float32),
        pltpu.SemaphoreType.DMA,
        pltpu.SemaphoreType.DMA,
        pltpu.SemaphoreType.DMA,
        pltpu.SemaphoreType.DMA,
    ]
    if with_deg:
        out_type.append(jax.ShapeDtypeStruct((NC, NACC, DW), jnp.float32))
        scratch.append(pltpu.VMEM_SHARED((NACC, DW), jnp.float32))
        scratch.append(pltpu.VMEM((CHUNK, DW), jnp.float32))

    @functools.partial(
        pl.kernel,
        out_type=out_type,
        mesh=mesh,
        scratch_types=scratch,
        compiler_params=pltpu.CompilerParams(use_tc_tiling_on_sc=False),
    )
    def sc_agg(table, srcr, dstr, *rest):
        if with_deg:
            (parts, degp, acc, sidx, didx, buf0, buf1,
             gsem0, gsem1, isem_s, isem_d, dacc, onesb) = rest
        else:
            (parts, acc, sidx, didx, buf0, buf1,
             gsem0, gsem1, isem_s, isem_d) = rest
        c = lax.axis_index("c")
        s = lax.axis_index("s")
        wid = s * NC + c
        mysrc = srcr.at[wid]
        mydst = dstr.at[wid]

        @pl.loop(0, CHUNK)
        def _(r):
            for cp in range(D // 16):
                buf0[r, pl.ds(cp * 16, 16)] = jnp.zeros((16,), jnp.float32)
            if with_deg:
                onesb[r, pl.ds(0, 16)] = jnp.zeros((16,), jnp.float32)

        off = 0
        while off < PT:
            sz = min(CHUNK, PT - off)
            pltpu.sync_copy(buf0.at[pl.ds(0, sz)],
                            acc.at[pl.ds(s * PT + off, sz)])
            if with_deg:
                pltpu.sync_copy(onesb.at[pl.ds(0, sz)],
                                dacc.at[pl.ds(s * PT + off, sz)])
            off += sz

        if with_deg:
            @pl.loop(0, CHUNK)
            def _(r):
                onesb[r, pl.ds(0, 16)] = jnp.ones((16,), jnp.float32)

        pltpu.sync_copy(mysrc.at[pl.ds(0, GP)], sidx.at[pl.ds(0, GP)])
        pltpu.sync_copy(mydst.at[pl.ds(0, GP)], didx.at[pl.ds(0, GP)])
        pltpu.async_copy(mysrc.at[pl.ds(GP, GP)], sidx.at[pl.ds(GP, GP)],
                         isem_s)
        pltpu.async_copy(mydst.at[pl.ds(GP, GP)], didx.at[pl.ds(GP, GP)],
                         isem_d)
        plsc.subcore_barrier()

        pltpu.async_copy(table.at[sidx.at[0]], buf0, gsem0)
        pltpu.async_copy(table.at[sidx.at[1]], buf1, gsem1)

        @pl.loop(0, NG)
        def _(g):
            base = lax.rem(g, 2) * GP
            nbase = lax.rem(g + 1, 2) * GP

            @pl.when(g < NG - 1)
            def _():
                pltpu.make_async_copy(mysrc.at[pl.ds(0, GP)],
                                      sidx.at[pl.ds(0, GP)], isem_s).wait()
                pltpu.make_async_copy(mydst.at[pl.ds(0, GP)],
                                      didx.at[pl.ds(0, GP)], isem_d).wait()

            for k in range(GP):
                buf, gsem = (buf0, gsem0) if k % 2 == 0 else (buf1, gsem1)
                pltpu.make_async_copy(table.at[sidx.at[0]], buf, gsem).wait()
                pltpu.sync_copy(buf, acc.at[didx.at[base + k]], add=True)
                if with_deg:
                    pltpu.sync_copy(onesb, dacc.at[didx.at[base + k]],
                                    add=True)
                if k < 2:
                    pltpu.async_copy(table.at[sidx.at[base + k + 2]], buf,
                                     gsem)
                else:
                    @pl.when(g < NG - 1)
                    def _():
                        pltpu.async_copy(table.at[sidx.at[nbase + k - 2]],
                                         buf, gsem)

            @pl.when(g < NG - 2)
            def _():
                pltpu.async_copy(mysrc.at[pl.ds((g + 2) * GP, GP)],
                                 sidx.at[pl.ds(base, GP)], isem_s)
                pltpu.async_copy(mydst.at[pl.ds((g + 2) * GP, GP)],
                                 didx.at[pl.ds(base, GP)], isem_d)

        plsc.subcore_barrier()
        pltpu.sync_copy(acc.at[pl.ds(s * PT, PT)],
                        parts.at[c].at[pl.ds(s * PT, PT)])
        if with_deg:
            pltpu.sync_copy(dacc.at[pl.ds(s * PT, PT)],
                            degp.at[c].at[pl.ds(s * PT, PT)])

    return sc_agg


def _xside_body(x, w, b, out_ref):
    out_ref[...] = lax.dot_general(
        x[...], w[...], (((1,), (1,)), ((), ())),
        preferred_element_type=jnp.float32,
        precision=lax.Precision.HIGHEST) + b[...]


_DNT = (((1,), (1,)), ((), ()))
_DN = (((1,), (0,)), ((), ()))


def _combine0_body(p0, p1, d0, d1, dsel, wl, zr, h_ref, rdeg_ref):
    ps = p0[...] + p1[...]
    zl = lax.dot_general(ps, wl[...], _DNT,
                         preferred_element_type=jnp.float32,
                         precision=lax.Precision.HIGHEST)
    degb = lax.dot_general(d0[...] + d1[...], dsel[...], _DN,
                           preferred_element_type=jnp.float32,
                           precision=lax.Precision.HIGHEST)
    rdeg = 1.0 / jnp.maximum(degb, 1.0)
    z = zl * rdeg + zr[...]
    sig = 1.0 / (1.0 + jnp.exp(-z))
    h_ref[...] = z * (0.5 + 0.5 * sig)
    rdeg_ref[...] = rdeg


def _combine1_body(p0, p1, rdeg, wl, zr, out_ref):
    ps = p0[...] + p1[...]
    zl = lax.dot_general(ps, wl[...], _DNT,
                         preferred_element_type=jnp.float32,
                         precision=lax.Precision.HIGHEST)
    out_ref[...] = zl * rdeg[...] + zr[...]


_RB = 2000
_GRID = N // _RB


def _row_spec(width):
    return pl.BlockSpec((_RB, width), lambda i: (i, 0))


def _full_spec(shape):
    return pl.BlockSpec(shape, lambda i: tuple(0 for _ in shape))


def _xside(x, w, b):
    return pl.pallas_call(
        _xside_body,
        grid=(_GRID,),
        in_specs=[_row_spec(D), _full_spec((D, D)), _full_spec((1, D))],
        out_specs=_row_spec(D),
        out_shape=jax.ShapeDtypeStruct((N, D), jnp.float32),
    )(x, w, b)


def _combine0(p0, p1, d0, d1, dsel, wl, zr):
    return pl.pallas_call(
        _combine0_body,
        grid=(_GRID,),
        in_specs=[
            _row_spec(D), _row_spec(D), _row_spec(DW), _row_spec(DW),
            _full_spec((DW, D)), _full_spec((D, D)), _row_spec(D),
        ],
        out_specs=[_row_spec(D), _row_spec(D)],
        out_shape=[jax.ShapeDtypeStruct((N, D), jnp.float32),
                   jax.ShapeDtypeStruct((N, D), jnp.float32)],
    )(p0, p1, d0, d1, dsel, wl, zr)


def _combine1(p0, p1, rdeg, wl, zr):
    return pl.pallas_call(
        _combine1_body,
        grid=(_GRID,),
        in_specs=[
            _row_spec(D), _row_spec(D), _row_spec(D),
            _full_spec((D, D)), _row_spec(D),
        ],
        out_specs=_row_spec(D),
        out_shape=jax.ShapeDtypeStruct((N, D), jnp.float32),
    )(p0, p1, rdeg, wl, zr)


def kernel(x, edge_index, W_l0, b_l0, W_r0, W_l1, b_l1, W_r1):
    src = edge_index[0]
    dst = edge_index[1]
    pad = EP - E
    pad_iota = jnp.arange(pad, dtype=jnp.int32)
    srcp = jnp.concatenate([src, pad_iota % N]).reshape(NW, CH, CHUNK)
    dstp = jnp.concatenate([dst, N + pad_iota % (NACC - N)]).reshape(
        NW, CH, CHUNK)

    parts0, degp = _make_sc_agg(True)(x, srcp, dstp)
    zr0 = _xside(x, W_r0, b_l0.reshape(1, D))

    dsel = jnp.full((DW, D), 1.0 / DW, jnp.float32)
    h0, rdeg = _combine0(parts0[0], parts0[1], degp[0], degp[1], dsel,
                         W_l0, zr0)

    parts1, = _make_sc_agg(False)(h0, srcp, dstp)
    zr1 = _xside(h0, W_r1, b_l1.reshape(1, D))

    return _combine1(parts1[0], parts1[1], rdeg, W_l1, zr1)


# speedup vs baseline: 1.4114x; 1.0747x over previous
import functools

import jax
import jax.numpy as jnp
from jax import lax
from jax.experimental import pallas as pl
from jax.experimental.pallas import tpu as pltpu
from jax.experimental.pallas import tpu_sc as plsc

N = 10000
D = 128
E = 320000

NC = 2
NS = 16
NW = NC * NS
CHUNK = 128
CH = 80
GP = 4
NG = CH // GP
EP = NW * CH * CHUNK
NACC = 10240
PT = NACC // NS
DW = 16


@functools.lru_cache(maxsize=None)
def _make_sc_agg(with_deg):
    mesh = plsc.VectorSubcoreMesh(core_axis_name="c", subcore_axis_name="s")

    out_type = [jax.ShapeDtypeStruct((NC, NACC, D), jnp.float32)]
    scratch = [
        pltpu.VMEM_SHARED((NACC, D), jnp.float32),
        pltpu.VMEM((2 * GP, CHUNK), jnp.int32),
        pltpu.VMEM((2 * GP, CHUNK), jnp.int32),
        pltpu.VMEM((CHUNK, D), jnp.float32),
        pltpu.VMEM((CHUNK, D), jnp.float32),
        pltpu.SemaphoreType.DMA,
        pltpu.SemaphoreType.DMA,
        pltpu.SemaphoreType.DMA,
        pltpu.SemaphoreType.DMA,
    ]
    if with_deg:
        out_type.append(jax.ShapeDtypeStruct((NC, NACC, DW), jnp.float32))
        scratch.append(pltpu.VMEM_SHARED((NACC, DW), jnp.float32))
        scratch.append(pltpu.VMEM((CHUNK, DW), jnp.float32))

    @functools.partial(
        pl.kernel,
        out_type=out_type,
        mesh=mesh,
        scratch_types=scratch,
        compiler_params=pltpu.CompilerParams(use_tc_tiling_on_sc=False),
    )
    def sc_agg(table, srcr, dstr, *rest):
        if with_deg:
            (parts, degp, acc, sidx, didx, buf0, buf1,
             gsem0, gsem1, isem_s, isem_d, dacc, onesb) = rest
        else:
            (parts, acc, sidx, didx, buf0, buf1,
             gsem0, gsem1, isem_s, isem_d) = rest
        c = lax.axis_index("c")
        s = lax.axis_index("s")
        wid = s * NC + c
        mysrc = srcr.at[wid]
        mydst = dstr.at[wid]

        @pl.loop(0, CHUNK)
        def _(r):
            for cp in range(D // 16):
                buf0[r, pl.ds(cp * 16, 16)] = jnp.zeros((16,), jnp.float32)
            if with_deg:
                onesb[r, pl.ds(0, 16)] = jnp.zeros((16,), jnp.float32)

        off = 0
        while off < PT:
            sz = min(CHUNK, PT - off)
            pltpu.sync_copy(buf0.at[pl.ds(0, sz)],
                            acc.at[pl.ds(s * PT + off, sz)])
            if with_deg:
                pltpu.sync_copy(onesb.at[pl.ds(0, sz)],
                                dacc.at[pl.ds(s * PT + off, sz)])
            off += sz

        if with_deg:
            @pl.loop(0, CHUNK)
            def _(r):
                onesb[r, pl.ds(0, 16)] = jnp.ones((16,), jnp.float32)

        pltpu.sync_copy(mysrc.at[pl.ds(0, GP)], sidx.at[pl.ds(0, GP)])
        pltpu.sync_copy(mydst.at[pl.ds(0, GP)], didx.at[pl.ds(0, GP)])
        pltpu.async_copy(mysrc.at[pl.ds(GP, GP)], sidx.at[pl.ds(GP, GP)],
                         isem_s)
        pltpu.async_copy(mydst.at[pl.ds(GP, GP)], didx.at[pl.ds(GP, GP)],
                         isem_d)
        plsc.subcore_barrier()

        pltpu.async_copy(table.at[sidx.at[0]], buf0, gsem0)
        pltpu.async_copy(table.at[sidx.at[1]], buf1, gsem1)

        @pl.loop(0, NG)
        def _(g):
            base = lax.rem(g, 2) * GP
            nbase = lax.rem(g + 1, 2) * GP

            @pl.when(g < NG - 1)
            def _():
                pltpu.make_async_copy(mysrc.at[pl.ds(0, GP)],
                                      sidx.at[pl.ds(0, GP)], isem_s).wait()
                pltpu.make_async_copy(mydst.at[pl.ds(0, GP)],
                                      didx.at[pl.ds(0, GP)], isem_d).wait()

            for k in range(GP):
                buf, gsem = (buf0, gsem0) if k % 2 == 0 else (buf1, gsem1)
                pltpu.make_async_copy(table.at[sidx.at[0]], buf, gsem).wait()
                pltpu.sync_copy(buf, acc.at[didx.at[base + k]], add=True)
                if with_deg:
                    pltpu.sync_copy(onesb, dacc.at[didx.at[base + k]],
                                    add=True)
                if k < 2:
                    pltpu.async_copy(table.at[sidx.at[base + k + 2]], buf,
                                     gsem)
                else:
                    @pl.when(g < NG - 1)
                    def _():
                        pltpu.async_copy(table.at[sidx.at[nbase + k - 2]],
                                         buf, gsem)

            @pl.when(g < NG - 2)
            def _():
                pltpu.async_copy(mysrc.at[pl.ds((g + 2) * GP, GP)],
                                 sidx.at[pl.ds(base, GP)], isem_s)
                pltpu.async_copy(mydst.at[pl.ds((g + 2) * GP, GP)],
                                 didx.at[pl.ds(base, GP)], isem_d)

        plsc.subcore_barrier()
        pltpu.sync_copy(acc.at[pl.ds(s * PT, PT)],
                        parts.at[c].at[pl.ds(s * PT, PT)])
        if with_deg:
            pltpu.sync_copy(dacc.at[pl.ds(s * PT, PT)],
                            degp.at[c].at[pl.ds(s * PT, PT)])

    return sc_agg


def _xside_body(x, w, b, out_ref):
    out_ref[...] = lax.dot_general(
        x[...], w[...], (((1,), (1,)), ((), ())),
        preferred_element_type=jnp.float32,
        precision=lax.Precision.HIGHEST) + b[...]


_DNT = (((1,), (1,)), ((), ()))
_DN = (((1,), (0,)), ((), ()))


def _combine0_body(p, d, dsel, wl, zr, h_ref, rdeg_ref):
    ps = p[0] + p[1]
    zl = lax.dot_general(ps, wl[...], _DNT,
                         preferred_element_type=jnp.float32,
                         precision=lax.Precision.HIGHEST)
    dsum = d[0] + d[1]
    degb = lax.dot_general(dsum, dsel[...], _DN,
                           preferred_element_type=jnp.float32,
                           precision=lax.Precision.HIGHEST)
    rdeg = 1.0 / jnp.maximum(degb, 1.0)
    z = zl * rdeg + zr[...]
    sig = 1.0 / (1.0 + jnp.exp(-z))
    h_ref[...] = z * (0.5 + 0.5 * sig)
    rdeg_ref[...] = rdeg


def _combine1_body(p, rdeg, wl, zr, out_ref):
    ps = p[0] + p[1]
    zl = lax.dot_general(ps, wl[...], _DNT,
                         preferred_element_type=jnp.float32,
                         precision=lax.Precision.HIGHEST)
    out_ref[...] = zl * rdeg[...] + zr[...]


_RB = 2048
_GRID = NACC // _RB


def _row_spec(width):
    return pl.BlockSpec((_RB, width), lambda i: (i, 0))


def _full_spec(shape):
    return pl.BlockSpec(shape, lambda i: tuple(0 for _ in shape))


def _xside(x, w, b):
    return pl.pallas_call(
        _xside_body,
        grid=(_GRID,),
        in_specs=[_row_spec(D), _full_spec((D, D)), _full_spec((1, D))],
        out_specs=_row_spec(D),
        out_shape=jax.ShapeDtypeStruct((N, D), jnp.float32),
    )(x, w, b)


_DRB = _RB * DW // D


def _pair_spec(rows, width):
    return pl.BlockSpec((NC, rows, width), lambda i: (0, i, 0))


def _combine0(p, dflat, dsel, wl, zr):
    return pl.pallas_call(
        _combine0_body,
        grid=(_GRID,),
        in_specs=[
            _pair_spec(_RB, D), _pair_spec(_RB, DW),
            _full_spec((DW, D)), _full_spec((D, D)), _row_spec(D),
        ],
        out_specs=[_row_spec(D), _row_spec(D)],
        out_shape=[jax.ShapeDtypeStruct((N, D), jnp.float32),
                   jax.ShapeDtypeStruct((N, D), jnp.float32)],
    )(p, dflat, dsel, wl, zr)


def _combine1(p, rdeg, wl, zr):
    return pl.pallas_call(
        _combine1_body,
        grid=(_GRID,),
        in_specs=[
            _pair_spec(_RB, D), _row_spec(D),
            _full_spec((D, D)), _row_spec(D),
        ],
        out_specs=_row_spec(D),
        out_shape=jax.ShapeDtypeStruct((N, D), jnp.float32),
    )(p, rdeg, wl, zr)


def kernel(x, edge_index, W_l0, b_l0, W_r0, W_l1, b_l1, W_r1):
    src = edge_index[0]
    dst = edge_index[1]
    pad = EP - E
    pad_iota = jnp.arange(pad, dtype=jnp.int32)
    srcp = jnp.concatenate([src, pad_iota % N]).reshape(NW, CH, CHUNK)
    dstp = jnp.concatenate([dst, N + pad_iota % (NACC - N)]).reshape(
        NW, CH, CHUNK)

    parts0, degp = _make_sc_agg(True)(x, srcp, dstp)
    zr0 = _xside(x, W_r0, b_l0.reshape(1, D))

    dsel = jnp.full((DW, D), 1.0 / DW, jnp.float32)
    h0, rdeg = _combine0(parts0, degp, dsel, W_l0, zr0)

    parts1, = _make_sc_agg(False)(h0, srcp, dstp)
    zr1 = _xside(h0, W_r1, b_l1.reshape(1, D))

    return _combine1(parts1, rdeg, W_l1, zr1)


# speedup vs baseline: 1.4809x; 1.0492x over previous
import functools

import jax
import jax.numpy as jnp
from jax import lax
from jax.experimental import pallas as pl
from jax.experimental.pallas import tpu as pltpu
from jax.experimental.pallas import tpu_sc as plsc

N = 10000
D = 128
E = 320000

NC = 2
NS = 16
NW = NC * NS
CHUNK = 128
NCHT = E // CHUNK
CH = 80
GP = 4
NG = CH // GP
NACC = 10240
PT = NACC // NS
DW = 16


@functools.lru_cache(maxsize=None)
def _make_sc_agg(with_deg):
    mesh = plsc.VectorSubcoreMesh(core_axis_name="c", subcore_axis_name="s")

    out_type = [jax.ShapeDtypeStruct((NC, NACC, D), jnp.float32)]
    scratch = [
        pltpu.VMEM_SHARED((NACC, D), jnp.float32),
        pltpu.VMEM((2 * GP, 2, CHUNK), jnp.int32),
        pltpu.VMEM((CHUNK, D), jnp.float32),
        pltpu.VMEM((CHUNK, D), jnp.float32),
        pltpu.SemaphoreType.DMA,
        pltpu.SemaphoreType.DMA,
        pltpu.SemaphoreType.DMA,
    ]
    if with_deg:
        out_type.append(jax.ShapeDtypeStruct((NC, NACC, DW), jnp.float32))
        scratch.append(pltpu.VMEM_SHARED((NACC, DW), jnp.float32))
        scratch.append(pltpu.VMEM((CHUNK, DW), jnp.float32))

    @functools.partial(
        pl.kernel,
        out_type=out_type,
        mesh=mesh,
        scratch_types=scratch,
        compiler_params=pltpu.CompilerParams(use_tc_tiling_on_sc=False),
    )
    def sc_agg(table, esd, *rest):
        if with_deg:
            (parts, degp, acc, ring, buf0, buf1,
             gsem0, gsem1, isem, dacc, onesb) = rest
        else:
            (parts, acc, ring, buf0, buf1, gsem0, gsem1, isem) = rest
        c = lax.axis_index("c")
        s = lax.axis_index("s")
        wid = s * NC + c
        cb = wid * CH
        ngw = jnp.where(wid == NW - 1, (NCHT - (NW - 1) * CH) // GP, NG)

        @pl.loop(0, CHUNK)
        def _(r):
            for cp in range(D // 16):
                buf0[r, pl.ds(cp * 16, 16)] = jnp.zeros((16,), jnp.float32)
            if with_deg:
                onesb[r, pl.ds(0, 16)] = jnp.zeros((16,), jnp.float32)

        off = 0
        while off < PT:
            sz = min(CHUNK, PT - off)
            pltpu.sync_copy(buf0.at[pl.ds(0, sz)],
                            acc.at[pl.ds(s * PT + off, sz)])
            if with_deg:
                pltpu.sync_copy(onesb.at[pl.ds(0, sz)],
                                dacc.at[pl.ds(s * PT + off, sz)])
            off += sz

        if with_deg:
            @pl.loop(0, CHUNK)
            def _(r):
                onesb[r, pl.ds(0, 16)] = jnp.ones((16,), jnp.float32)

        pltpu.sync_copy(esd.at[pl.ds(cb, GP)], ring.at[pl.ds(0, GP)])
        pltpu.async_copy(esd.at[pl.ds(cb + GP, GP)], ring.at[pl.ds(GP, GP)],
                         isem)
        plsc.subcore_barrier()

        pltpu.async_copy(table.at[ring.at[0, 0]], buf0, gsem0)
        pltpu.async_copy(table.at[ring.at[1, 0]], buf1, gsem1)

        @pl.loop(0, NG)
        def _(g):
            @pl.when(g < ngw)
            def _():
                base = lax.rem(g, 2) * GP
                nbase = lax.rem(g + 1, 2) * GP

                @pl.when(g < ngw - 1)
                def _():
                    pltpu.make_async_copy(esd.at[pl.ds(cb, GP)],
                                          ring.at[pl.ds(0, GP)], isem).wait()

                for k in range(GP):
                    buf, gsem = (buf0, gsem0) if k % 2 == 0 else (buf1, gsem1)
                    pltpu.make_async_copy(table.at[ring.at[0, 0]], buf,
                                          gsem).wait()
                    pltpu.sync_copy(buf, acc.at[ring.at[base + k, 1]],
                                    add=True)
                    if with_deg:
                        pltpu.sync_copy(onesb, dacc.at[ring.at[base + k, 1]],
                                        add=True)
                    if k < 2:
                        pltpu.async_copy(table.at[ring.at[base + k + 2, 0]],
                                         buf, gsem)
                    else:
                        @pl.when(g < ngw - 1)
                        def _():
                            pltpu.async_copy(
                                table.at[ring.at[nbase + k - 2, 0]], buf,
                                gsem)

                @pl.when(g < ngw - 2)
                def _():
                    pltpu.async_copy(esd.at[pl.ds(cb + (g + 2) * GP, GP)],
                                     ring.at[pl.ds(base, GP)], isem)

        plsc.subcore_barrier()
        pltpu.sync_copy(acc.at[pl.ds(s * PT, PT)],
                        parts.at[c].at[pl.ds(s * PT, PT)])
        if with_deg:
            pltpu.sync_copy(dacc.at[pl.ds(s * PT, PT)],
                            degp.at[c].at[pl.ds(s * PT, PT)])

    return sc_agg


def _xside_body(x, w, b, out_ref):
    out_ref[...] = lax.dot_general(
        x[...], w[...], (((1,), (1,)), ((), ())),
        preferred_element_type=jnp.float32,
        precision=lax.Precision.HIGHEST) + b[...]


_DNT = (((1,), (1,)), ((), ()))
_DN = (((1,), (0,)), ((), ()))


def _combine0_body(p, d, dsel, wl, zr, h_ref, rdeg_ref):
    ps = p[0] + p[1]
    zl = lax.dot_general(ps, wl[...], _DNT,
                         preferred_element_type=jnp.float32,
                         precision=lax.Precision.HIGHEST)
    dsum = d[0] + d[1]
    degb = lax.dot_general(dsum, dsel[...], _DN,
                           preferred_element_type=jnp.float32,
                           precision=lax.Precision.HIGHEST)
    rdeg = 1.0 / jnp.maximum(degb, 1.0)
    z = zl * rdeg + zr[...]
    sig = 1.0 / (1.0 + jnp.exp(-z))
    h_ref[...] = z * (0.5 + 0.5 * sig)
    rdeg_ref[...] = rdeg


def _combine1_body(p, rdeg, wl, zr, out_ref):
    ps = p[0] + p[1]
    zl = lax.dot_general(ps, wl[...], _DNT,
                         preferred_element_type=jnp.float32,
                         precision=lax.Precision.HIGHEST)
    out_ref[...] = zl * rdeg[...] + zr[...]


_RB = 2048
_GRID = NACC // _RB


def _row_spec(width):
    return pl.BlockSpec((_RB, width), lambda i: (i, 0))


def _full_spec(shape):
    return pl.BlockSpec(shape, lambda i: tuple(0 for _ in shape))


def _xside(x, w, b):
    return pl.pallas_call(
        _xside_body,
        grid=(_GRID,),
        in_specs=[_row_spec(D), _full_spec((D, D)), _full_spec((1, D))],
        out_specs=_row_spec(D),
        out_shape=jax.ShapeDtypeStruct((N, D), jnp.float32),
    )(x, w, b)


_DRB = _RB * DW // D


def _pair_spec(rows, width):
    return pl.BlockSpec((NC, rows, width), lambda i: (0, i, 0))


def _combine0(p, dflat, dsel, wl, zr):
    return pl.pallas_call(
        _combine0_body,
        grid=(_GRID,),
        in_specs=[
            _pair_spec(_RB, D), _pair_spec(_RB, DW),
            _full_spec((DW, D)), _full_spec((D, D)), _row_spec(D),
        ],
        out_specs=[_row_spec(D), _row_spec(D)],
        out_shape=[jax.ShapeDtypeStruct((N, D), jnp.float32),
                   jax.ShapeDtypeStruct((N, D), jnp.float32)],
    )(p, dflat, dsel, wl, zr)


def _combine1(p, rdeg, wl, zr):
    return pl.pallas_call(
        _combine1_body,
        grid=(_GRID,),
        in_specs=[
            _pair_spec(_RB, D), _row_spec(D),
            _full_spec((D, D)), _row_spec(D),
        ],
        out_specs=_row_spec(D),
        out_shape=jax.ShapeDtypeStruct((N, D), jnp.float32),
    )(p, rdeg, wl, zr)


def kernel(x, edge_index, W_l0, b_l0, W_r0, W_l1, b_l1, W_r1):
    esd = jnp.transpose(edge_index.reshape(2, NCHT, CHUNK), (1, 0, 2))

    parts0, degp = _make_sc_agg(True)(x, esd)
    zr0 = _xside(x, W_r0, b_l0.reshape(1, D))

    dsel = jnp.full((DW, D), 1.0 / DW, jnp.float32)
    h0, rdeg = _combine0(parts0, degp, dsel, W_l0, zr0)

    parts1, = _make_sc_agg(False)(h0, esd)
    zr1 = _xside(h0, W_r1, b_l1.reshape(1, D))

    return _combine1(parts1, rdeg, W_l1, zr1)


# speedup vs baseline: 1.4852x; 1.0029x over previous
import functools

import jax
import jax.numpy as jnp
from jax import lax
from jax.experimental import pallas as pl
from jax.experimental.pallas import tpu as pltpu
from jax.experimental.pallas import tpu_sc as plsc

N = 10000
D = 128
E = 320000

NC = 2
NS = 16
NW = NC * NS
CHUNK = 128
NCHT = E // CHUNK
CH = 80
GP = 4
NG = CH // GP
NACC = 10240
PT = NACC // NS
DW = 16


@functools.lru_cache(maxsize=None)
def _make_sc_agg(with_deg):
    mesh = plsc.VectorSubcoreMesh(core_axis_name="c", subcore_axis_name="s")

    out_type = [jax.ShapeDtypeStruct((NC, NACC, D), jnp.float32)]
    scratch = [
        pltpu.VMEM_SHARED((NACC, D), jnp.float32),
        pltpu.VMEM((2 * GP, 2, CHUNK), jnp.int32),
        pltpu.VMEM((CHUNK, D), jnp.float32),
        pltpu.VMEM((CHUNK, D), jnp.float32),
        pltpu.SemaphoreType.DMA,
        pltpu.SemaphoreType.DMA,
        pltpu.SemaphoreType.DMA,
    ]
    if with_deg:
        out_type.append(jax.ShapeDtypeStruct((NC, NACC, DW), jnp.float32))
        scratch.append(pltpu.VMEM_SHARED((NACC, DW), jnp.float32))
        scratch.append(pltpu.VMEM((CHUNK, DW), jnp.float32))

    @functools.partial(
        pl.kernel,
        out_type=out_type,
        mesh=mesh,
        scratch_types=scratch,
        compiler_params=pltpu.CompilerParams(use_tc_tiling_on_sc=False),
    )
    def sc_agg(table, esd, *rest):
        if with_deg:
            (parts, degp, acc, ring, buf0, buf1,
             gsem0, gsem1, isem, dacc, onesb) = rest
        else:
            (parts, acc, ring, buf0, buf1, gsem0, gsem1, isem) = rest
        c = lax.axis_index("c")
        s = lax.axis_index("s")
        wid = s * NC + c
        cb = wid * CH
        ngw = jnp.where(wid == NW - 1, (NCHT - (NW - 1) * CH) // GP, NG)

        @pl.loop(0, CHUNK)
        def _(r):
            for cp in range(D // 16):
                buf0[r, pl.ds(cp * 16, 16)] = jnp.zeros((16,), jnp.float32)
            if with_deg:
                onesb[r, pl.ds(0, 16)] = jnp.zeros((16,), jnp.float32)

        off = 0
        while off < PT:
            sz = min(CHUNK, PT - off)
            pltpu.sync_copy(buf0.at[pl.ds(0, sz)],
                            acc.at[pl.ds(s * PT + off, sz)])
            if with_deg:
                pltpu.sync_copy(onesb.at[pl.ds(0, sz)],
                                dacc.at[pl.ds(s * PT + off, sz)])
            off += sz

        if with_deg:
            @pl.loop(0, CHUNK)
            def _(r):
                onesb[r, pl.ds(0, 16)] = jnp.ones((16,), jnp.float32)

        pltpu.sync_copy(esd.at[pl.ds(cb, GP)], ring.at[pl.ds(0, GP)])
        pltpu.async_copy(esd.at[pl.ds(cb + GP, GP)], ring.at[pl.ds(GP, GP)],
                         isem)
        plsc.subcore_barrier()

        pltpu.async_copy(table.at[ring.at[0, 0]], buf0, gsem0)
        pltpu.async_copy(table.at[ring.at[1, 0]], buf1, gsem1)

        @pl.loop(0, NG)
        def _(g):
            @pl.when(g < ngw)
            def _():
                base = lax.rem(g, 2) * GP
                nbase = lax.rem(g + 1, 2) * GP

                @pl.when(g < ngw - 1)
                def _():
                    pltpu.make_async_copy(esd.at[pl.ds(cb, GP)],
                                          ring.at[pl.ds(0, GP)], isem).wait()

                for k in range(GP):
                    buf, gsem = (buf0, gsem0) if k % 2 == 0 else (buf1, gsem1)
                    pltpu.make_async_copy(table.at[ring.at[0, 0]], buf,
                                          gsem).wait()
                    pltpu.sync_copy(buf, acc.at[ring.at[base + k, 1]],
                                    add=True)
                    if with_deg:
                        pltpu.sync_copy(onesb, dacc.at[ring.at[base + k, 1]],
                                        add=True)
                    if k < 2:
                        pltpu.async_copy(table.at[ring.at[base + k + 2, 0]],
                                         buf, gsem)
                    else:
                        @pl.when(g < ngw - 1)
                        def _():
                            pltpu.async_copy(
                                table.at[ring.at[nbase + k - 2, 0]], buf,
                                gsem)

                @pl.when(g < ngw - 2)
                def _():
                    pltpu.async_copy(esd.at[pl.ds(cb + (g + 2) * GP, GP)],
                                     ring.at[pl.ds(base, GP)], isem)

        plsc.subcore_barrier()
        pltpu.sync_copy(acc.at[pl.ds(s * PT, PT)],
                        parts.at[c].at[pl.ds(s * PT, PT)])
        if with_deg:
            pltpu.sync_copy(dacc.at[pl.ds(s * PT, PT)],
                            degp.at[c].at[pl.ds(s * PT, PT)])

    return sc_agg


def _xside_body(x, w, b, out_ref):
    out_ref[...] = lax.dot_general(
        x[...], w[...], (((1,), (1,)), ((), ())),
        preferred_element_type=jnp.float32,
        precision=lax.Precision.HIGHEST) + b[...]


_DNT = (((1,), (1,)), ((), ()))
_DN = (((1,), (0,)), ((), ()))


def _combine0_body(p, d, dsel, wl, zr, h_ref, rdeg_ref):
    ps = p[0] + p[1]
    zl = lax.dot_general(ps, wl[...], _DNT,
                         preferred_element_type=jnp.float32,
                         precision=lax.Precision.HIGHEST)
    dsum = d[0] + d[1]
    degb = lax.dot_general(dsum, dsel[...], _DN,
                           preferred_element_type=jnp.float32,
                           precision=lax.Precision.HIGHEST)
    rdeg = 1.0 / jnp.maximum(degb, 1.0)
    z = zl * rdeg + zr[...]
    sig = 1.0 / (1.0 + jnp.exp(-z))
    h_ref[...] = z * (0.5 + 0.5 * sig)
    rdeg_ref[...] = rdeg


def _combine1_body(p, rdeg, wl, zr, out_ref):
    ps = p[0] + p[1]
    zl = lax.dot_general(ps, wl[...], _DNT,
                         preferred_element_type=jnp.float32,
                         precision=lax.Precision.HIGHEST)
    out_ref[...] = zl * rdeg[...] + zr[...]


_RB = 2048
_GRID = NACC // _RB


def _row_spec(width):
    return pl.BlockSpec((_RB, width), lambda i: (i, 0))


def _full_spec(shape):
    return pl.BlockSpec(shape, lambda i: tuple(0 for _ in shape))


def _xside(x, w, b):
    return pl.pallas_call(
        _xside_body,
        grid=(_GRID,),
        in_specs=[_row_spec(D), _full_spec((D, D)), _full_spec((1, D))],
        out_specs=_row_spec(D),
        out_shape=jax.ShapeDtypeStruct((N, D), jnp.float32),
    )(x, w, b)


def _pair_spec(rows, width):
    return pl.BlockSpec((NC, rows, width), lambda i: (0, i, 0))


def _combine0(p, dflat, dsel, wl, zr):
    return pl.pallas_call(
        _combine0_body,
        grid=(_GRID,),
        in_specs=[
            _pair_spec(_RB, D), _pair_spec(_RB, DW),
            _full_spec((DW, D)), _full_spec((D, D)), _row_spec(D),
        ],
        out_specs=[_row_spec(D), _row_spec(D)],
        out_shape=[jax.ShapeDtypeStruct((N, D), jnp.float32),
                   jax.ShapeDtypeStruct((N, D), jnp.float32)],
    )(p, dflat, dsel, wl, zr)


def _combine1(p, rdeg, wl, zr):
    return pl.pallas_call(
        _combine1_body,
        grid=(_GRID,),
        in_specs=[
            _pair_spec(_RB, D), _row_spec(D),
            _full_spec((D, D)), _row_spec(D),
        ],
        out_specs=_row_spec(D),
        out_shape=jax.ShapeDtypeStruct((N, D), jnp.float32),
    )(p, rdeg, wl, zr)


def kernel(x, edge_index, W_l0, b_l0, W_r0, W_l1, b_l1, W_r1):
    esd = jnp.transpose(edge_index.reshape(2, NCHT, CHUNK), (1, 0, 2))

    parts0, degp = _make_sc_agg(True)(x, esd)
    zr0 = _xside(x, W_r0, b_l0.reshape(1, D))

    dsel = jnp.full((DW, D), 1.0 / DW, jnp.float32)
    h0, rdeg = _combine0(parts0, degp, dsel, W_l0, zr0)

    parts1, = _make_sc_agg(False)(h0, esd)
    zr1 = _xside(h0, W_r1, b_l1.reshape(1, D))

    return _combine1(parts1, rdeg, W_l1, zr1)
